# baseline (speedup 1.0000x reference)
import jax
import jax.numpy as jnp
from jax.experimental import pallas as pl
from jax.experimental.pallas import tpu as pltpu

E_TOT = 32 * 1024
DIM = 128


def _copy_kernel(e_ref, o_ref, sem):
    pltpu.make_async_copy(e_ref, o_ref, sem).start()
    pltpu.make_async_copy(e_ref, o_ref, sem).wait()


def kernel(h, e, params, edge_index):
    out = pl.pallas_call(
        _copy_kernel,
        in_specs=[pl.BlockSpec(memory_space=pltpu.MemorySpace.HBM)],
        out_specs=pl.BlockSpec(memory_space=pltpu.MemorySpace.HBM),
        out_shape=jax.ShapeDtypeStruct((E_TOT, DIM), e.dtype),
        scratch_shapes=[pltpu.SemaphoreType.DMA],
    )(e)
    return out


# speedup vs baseline: 41.1592x; 41.1592x over previous
import jax
import jax.numpy as jnp
from jax.experimental import pallas as pl

E_TOT = 32 * 1024
DIM = 128
BLOCK_ROWS = 8192


def _copy_block(e_ref, o_ref):
    o_ref[...] = e_ref[...]


def kernel(h, e, params, edge_index):
    grid = (E_TOT // BLOCK_ROWS,)
    out = pl.pallas_call(
        _copy_block,
        grid=grid,
        in_specs=[pl.BlockSpec((BLOCK_ROWS, DIM), lambda i: (i, 0))],
        out_specs=pl.BlockSpec((BLOCK_ROWS, DIM), lambda i: (i, 0)),
        out_shape=jax.ShapeDtypeStruct((E_TOT, DIM), e.dtype),
    )(e)
    return out


# speedup vs baseline: 47.2852x; 1.1488x over previous
import jax
import jax.numpy as jnp
from jax.experimental import pallas as pl

E_TOT = 32 * 1024
DIM = 128
BLOCK_ROWS = 16384


def _copy_block(e_ref, o_ref):
    o_ref[...] = e_ref[...]


def kernel(h, e, params, edge_index):
    grid = (E_TOT // BLOCK_ROWS,)
    out = pl.pallas_call(
        _copy_block,
        grid=grid,
        in_specs=[pl.BlockSpec((BLOCK_ROWS, DIM), lambda i: (i, 0))],
        out_specs=pl.BlockSpec((BLOCK_ROWS, DIM), lambda i: (i, 0)),
        out_shape=jax.ShapeDtypeStruct((E_TOT, DIM), e.dtype),
    )(e)
    return out
